# baseline (speedup 1.0000x reference)
import functools
import math

import jax
import jax.numpy as jnp
from jax import lax
from jax.experimental import pallas as pl
from jax.experimental.pallas import tpu as pltpu
from jax.experimental.pallas import tpu_sc as plsc

VOCAB = 100000
DIM = 64
SEQ = 1500
BATCH = 1024
SEQ_PAD = 1504

NC = 2
NS = 16
NW = NC * NS
ROWS_PER_W = BATCH // NW

CL = 256
CHUNKS = ((0, 256), (256, 256), (512, 256), (768, 256), (1024, 256), (1280, 220))
IGS = 128
NBUF = 4


def _pe_table():
    position = jnp.arange(SEQ, dtype=jnp.float32)[:, None]
    div_term = jnp.exp(
        jnp.arange(0, DIM, 2, dtype=jnp.float32) * (-(math.log(10000.0) / DIM))
    )
    ang = position * div_term
    pe = jnp.zeros((SEQ, DIM), dtype=jnp.float32)
    pe = pe.at[:, 0::2].set(jnp.sin(ang))
    pe = pe.at[:, 1::2].set(jnp.cos(ang))
    return pe


@functools.partial(
    pl.kernel,
    mesh=plsc.VectorSubcoreMesh(core_axis_name="c", subcore_axis_name="s"),
    out_type=jax.ShapeDtypeStruct((BATCH, SEQ, DIM), jnp.float32),
    scratch_types=(
        [pltpu.VMEM((CL, DIM), jnp.float32)]
        + [pltpu.VMEM((CL,), jnp.int32) for _ in range(NBUF)]
        + [pltpu.VMEM((CL, DIM), jnp.float32) for _ in range(NBUF)]
        + [pltpu.SemaphoreType.DMA for _ in range(3 * NBUF)]
    ),
    compiler_params=pltpu.CompilerParams(use_tc_tiling_on_sc=False),
)
def _seq_encode(tok_hbm, pe_hbm, table_hbm, out_hbm, pe_v, *scratch):
    idx = scratch[:NBUF]
    rows = scratch[NBUF:2 * NBUF]
    sem_i = scratch[2 * NBUF:3 * NBUF]
    sem_g = scratch[3 * NBUF:4 * NBUF]
    sem_s = scratch[4 * NBUF:5 * NBUF]
    wid = lax.axis_index("s") * NC + lax.axis_index("c")

    for off, cl in CHUNKS:
        cl_pad = -(-cl // 8) * 8
        g2 = cl_pad - IGS
        unroll = 8 if cl % 8 == 0 else 4

        pltpu.sync_copy(pe_hbm.at[pl.ds(off, cl), :], pe_v.at[pl.ds(0, cl), :])

        def fire_idx(r, b, off=off, cl_pad=cl_pad):
            pltpu.async_copy(
                tok_hbm.at[pl.ds((wid * ROWS_PER_W + r) * SEQ_PAD + off, cl_pad)],
                idx[b].at[pl.ds(0, cl_pad)], sem_i[b])

        def drain_idx(b, cl_pad=cl_pad):
            pltpu.make_async_copy(
                tok_hbm.at[pl.ds(0, cl_pad)],
                idx[b].at[pl.ds(0, cl_pad)], sem_i[b]).wait()

        def fire_gathers(b, g2=g2):
            pltpu.async_copy(table_hbm.at[idx[b].at[pl.ds(0, IGS)]],
                             rows[b].at[pl.ds(0, IGS), :], sem_g[b])
            pltpu.async_copy(table_hbm.at[idx[b].at[pl.ds(IGS, g2)]],
                             rows[b].at[pl.ds(IGS, g2), :], sem_g[b])

        def drain_gathers(b, g2=g2):
            pltpu.make_async_copy(pe_hbm.at[pl.ds(0, IGS), :],
                                  rows[b].at[pl.ds(0, IGS), :], sem_g[b]).wait()
            pltpu.make_async_copy(pe_hbm.at[pl.ds(0, g2), :],
                                  rows[b].at[pl.ds(IGS, g2), :], sem_g[b]).wait()

        def add_pe(b, cl=cl, unroll=unroll):
            def add_body(i, c):
                for u in range(unroll):
                    p = i * unroll + u
                    for v in range(DIM // 16):
                        plsc.addupdate(rows[b].at[p, pl.ds(v * 16, 16)],
                                       pe_v[p, pl.ds(v * 16, 16)])
                return c
            lax.fori_loop(0, cl // unroll, add_body, 0)

        def fire_store(r, b, off=off, cl=cl):
            pltpu.async_copy(
                rows[b].at[pl.ds(0, cl), :],
                out_hbm.at[wid * ROWS_PER_W + r, pl.ds(off, cl), :], sem_s[b])

        def drain_store(b, off=off, cl=cl):
            pltpu.make_async_copy(
                rows[b].at[pl.ds(0, cl), :],
                out_hbm.at[0, pl.ds(off, cl), :], sem_s[b]).wait()

        fire_idx(0, 0)

        def slot_body(i, carry):
            for u in range(NBUF):
                r = i * NBUF + u
                pb = (u + NBUF - 1) % NBUF

                @pl.when(r >= 1)
                def _():
                    drain_gathers(pb)
                    add_pe(pb)
                    fire_store(r - 1, pb)

                @pl.when(r >= NBUF)
                def _():
                    drain_store(u)

                drain_idx(u)
                fire_gathers(u)

                @pl.when(r < ROWS_PER_W - 1)
                def _():
                    fire_idx(r + 1, (u + 1) % NBUF)
            return carry

        lax.fori_loop(0, ROWS_PER_W // NBUF, slot_body, 0)

        last_b = (ROWS_PER_W - 1) % NBUF
        drain_gathers(last_b)
        add_pe(last_b)
        fire_store(ROWS_PER_W - 1, last_b)
        for b in range(NBUF):
            drain_store(b)


def kernel(tokens, table):
    pe = _pe_table()
    tok_flat = jnp.pad(tokens, ((0, 0), (0, SEQ_PAD - SEQ))).reshape(-1)
    return _seq_encode(tok_flat, pe, table)
